# baseline (speedup 1.0000x reference)
import jax
import jax.numpy as jnp
from jax import lax
from jax.experimental import pallas as pl
from jax.experimental.pallas import tpu as pltpu
from jax.experimental.pallas import tpu_sc as plsc

NUM_USERS = 1000000
NUM_SUBS = 100000
D = 32
B = 16384
K = 20
W = 128
RPW = W // D

NC = 2
NS = 16
NW = NC * NS
BPW = B // NW
CHUNK = 32
NCHUNK = BPW // CHUNK
NEG_PER_CHUNK = CHUNK * K
GATHER_N = 128
NEG_GATHERS = NEG_PER_CHUNK // GATHER_N
GROUPS = CHUNK // 16


def _sc_body(uid_hbm, pid_hbm, nid_hbm, user_emb, sub_emb,
             pos_out, neg_out,
             idx_u, idx_p, idx_n, idx_u4, idx_p4, idx_n4,
             u_v, sp_v, sn_v, pos_v, neg_v, sem):
    wid = lax.axis_index("s") * NC + lax.axis_index("c")
    lanes = lax.iota(jnp.int32, 16)

    def chunk_body(c, carry):
        base = wid * BPW + c * CHUNK

        pltpu.sync_copy(uid_hbm.at[pl.ds(base, CHUNK)], idx_u)
        pltpu.sync_copy(pid_hbm.at[pl.ds(base, CHUNK)], idx_p)
        pltpu.sync_copy(nid_hbm.at[pl.ds(base * K, NEG_PER_CHUNK)], idx_n)

        def shift_body(i, _):
            idx_n4[pl.ds(i * 16, 16)] = idx_n[pl.ds(i * 16, 16)] >> 2
            return _
        lax.fori_loop(0, NEG_PER_CHUNK // 16, shift_body, 0)
        for i in range(CHUNK // 16):
            idx_u4[pl.ds(i * 16, 16)] = idx_u[pl.ds(i * 16, 16)] >> 2
            idx_p4[pl.ds(i * 16, 16)] = idx_p[pl.ds(i * 16, 16)] >> 2

        cps = [pltpu.async_copy(user_emb.at[idx_u4], u_v, sem),
               pltpu.async_copy(sub_emb.at[idx_p4], sp_v, sem)]
        for j in range(NEG_GATHERS):
            cps.append(pltpu.async_copy(
                sub_emb.at[idx_n4.at[pl.ds(j * GATHER_N, GATHER_N)]],
                sn_v.at[pl.ds(j * GATHER_N, GATHER_N), :], sem))
        for cp in cps:
            cp.wait()

        def group_body(g, gcarry):
            rows = g * 16 + lanes
            rows_k = rows * K
            zero = jnp.zeros((16,), jnp.float32)
            qu = (idx_u[pl.ds(g * 16, 16)] & (RPW - 1)) * D
            qp = (idx_p[pl.ds(g * 16, 16)] & (RPW - 1)) * D

            for half in range(2):
                ks = range(half * (K // 2), (half + 1) * (K // 2))
                qn = {k: (plsc.load_gather(idx_n, [rows_k + k]) & (RPW - 1)) * D
                      for k in ks}

                def d_body(d, accs):
                    diag = (lanes + d) & (D - 1)
                    u_d = plsc.load_gather(u_v, [rows, qu + diag])
                    out = []
                    if half == 0:
                        p_d = plsc.load_gather(sp_v, [rows, qp + diag])
                        out.append(accs[0] + u_d * p_d)
                    for i, k in enumerate(ks):
                        n_d = plsc.load_gather(sn_v, [rows_k + k, qn[k] + diag])
                        out.append(accs[i + (1 - half)] + u_d * n_d)
                    return tuple(out)

                n_acc = K // 2 + (1 - half)
                accs = lax.fori_loop(0, D, d_body, (zero,) * n_acc)
                if half == 0:
                    pos_v[pl.ds(g * 16, 16)] = accs[0]
                for i, k in enumerate(ks):
                    plsc.store_scatter(neg_v, [rows_k + k],
                                       accs[i + (1 - half)])
            return gcarry

        lax.fori_loop(0, GROUPS, group_body, 0)

        pltpu.sync_copy(pos_v, pos_out.at[pl.ds(base, CHUNK)])
        pltpu.sync_copy(neg_v, neg_out.at[pl.ds(base * K, NEG_PER_CHUNK)])
        return carry

    lax.fori_loop(0, NCHUNK, chunk_body, 0)


@jax.jit
def _sc_forward(user_ids, pos_sub_ids, neg_ids_flat, user_emb_w, sub_emb_w):
    mesh = plsc.VectorSubcoreMesh(core_axis_name="c", subcore_axis_name="s")
    return pl.kernel(
        _sc_body,
        out_type=(jax.ShapeDtypeStruct((B,), jnp.float32),
                  jax.ShapeDtypeStruct((B * K,), jnp.float32)),
        mesh=mesh,
        scratch_types=[
            pltpu.VMEM((CHUNK,), jnp.int32),
            pltpu.VMEM((CHUNK,), jnp.int32),
            pltpu.VMEM((NEG_PER_CHUNK,), jnp.int32),
            pltpu.VMEM((CHUNK,), jnp.int32),
            pltpu.VMEM((CHUNK,), jnp.int32),
            pltpu.VMEM((NEG_PER_CHUNK,), jnp.int32),
            pltpu.VMEM((CHUNK, W), jnp.float32),
            pltpu.VMEM((CHUNK, W), jnp.float32),
            pltpu.VMEM((NEG_PER_CHUNK, W), jnp.float32),
            pltpu.VMEM((CHUNK,), jnp.float32),
            pltpu.VMEM((NEG_PER_CHUNK,), jnp.float32),
            pltpu.SemaphoreType.DMA,
        ],
        compiler_params=pltpu.CompilerParams(use_tc_tiling_on_sc=False,
                                             needs_layout_passes=False),
    )(user_ids, pos_sub_ids, neg_ids_flat, user_emb_w, sub_emb_w)


def kernel(user_ids, pos_sub_ids, neg_sub_ids, user_emb, sub_emb):
    uid = user_ids.astype(jnp.int32)
    pid = pos_sub_ids.astype(jnp.int32)
    nid = neg_sub_ids.astype(jnp.int32).reshape(B * K)
    ue = user_emb.reshape(NUM_USERS * D // W, W)
    se = sub_emb.reshape(NUM_SUBS * D // W, W)
    pos_flat, neg_flat = _sc_forward(uid, pid, nid, ue, se)
    return (pos_flat, neg_flat.reshape(B, K))


# speedup vs baseline: 1.0218x; 1.0218x over previous
import jax
import jax.numpy as jnp
from jax import lax
from jax.experimental import pallas as pl
from jax.experimental.pallas import tpu as pltpu
from jax.experimental.pallas import tpu_sc as plsc

NUM_USERS = 1000000
NUM_SUBS = 100000
D = 32
B = 16384
K = 20
W = 128
RPW = W // D

NC = 2
NS = 16
NW = NC * NS
BPW = B // NW
CHUNK = 32
NCHUNK = BPW // CHUNK
NEG_PER_CHUNK = CHUNK * K
GATHER_N = 128
NEG_GATHERS = NEG_PER_CHUNK // GATHER_N
GROUPS = CHUNK // 16


def _sc_body(uid_hbm, pid_hbm, nid_hbm, user_emb, sub_emb,
             pos_out, neg_out,
             idx_u, idx_p, idx_n,
             u_v, sp_v, sn_v, pos_v, neg_v, sem):
    wid = lax.axis_index("s") * NC + lax.axis_index("c")
    lanes = lax.iota(jnp.int32, 16)

    def chunk_body(c, carry):
        base = wid * BPW + c * CHUNK

        pltpu.sync_copy(uid_hbm.at[pl.ds(base, CHUNK)], idx_u)
        pltpu.sync_copy(pid_hbm.at[pl.ds(base, CHUNK)], idx_p)
        pltpu.sync_copy(nid_hbm.at[pl.ds(base * K, NEG_PER_CHUNK)], idx_n)

        cps = [pltpu.async_copy(user_emb.at[idx_u], u_v, sem),
               pltpu.async_copy(sub_emb.at[idx_p], sp_v, sem)]
        for j in range(NEG_GATHERS):
            cps.append(pltpu.async_copy(
                sub_emb.at[idx_n.at[pl.ds(j * GATHER_N, GATHER_N)]],
                sn_v.at[pl.ds(j * GATHER_N, GATHER_N), :], sem))
        for cp in cps:
            cp.wait()

        def group_body(g, gcarry):
            rows = g * 16 + lanes
            rows_k = rows * K
            zero = jnp.zeros((16,), jnp.float32)
            def d_body(d, accs):
                diag = (lanes + d) & (D - 1)
                u_d = plsc.load_gather(u_v, [rows, diag])
                p_d = plsc.load_gather(sp_v, [rows, diag])
                out = [accs[0] + u_d * p_d]
                for k in range(K):
                    n_d = plsc.load_gather(sn_v, [rows_k + k, diag])
                    out.append(accs[k + 1] + u_d * n_d)
                return tuple(out)

            accs = lax.fori_loop(0, D, d_body, (zero,) * (K + 1))
            pos_v[pl.ds(g * 16, 16)] = accs[0]
            for k in range(K):
                plsc.store_scatter(neg_v, [rows_k + k], accs[k + 1])
            return gcarry

        lax.fori_loop(0, GROUPS, group_body, 0)

        pltpu.sync_copy(pos_v, pos_out.at[pl.ds(base, CHUNK)])
        pltpu.sync_copy(neg_v, neg_out.at[pl.ds(base * K, NEG_PER_CHUNK)])
        return carry

    lax.fori_loop(0, NCHUNK, chunk_body, 0)


@jax.jit
def _sc_forward(user_ids, pos_sub_ids, neg_ids_flat, user_emb_w, sub_emb_w):
    mesh = plsc.VectorSubcoreMesh(core_axis_name="c", subcore_axis_name="s")
    return pl.kernel(
        _sc_body,
        out_type=(jax.ShapeDtypeStruct((B,), jnp.float32),
                  jax.ShapeDtypeStruct((B * K,), jnp.float32)),
        mesh=mesh,
        scratch_types=[
            pltpu.VMEM((CHUNK,), jnp.int32),
            pltpu.VMEM((CHUNK,), jnp.int32),
            pltpu.VMEM((NEG_PER_CHUNK,), jnp.int32),
            pltpu.VMEM((CHUNK, W), jnp.float32),
            pltpu.VMEM((CHUNK, W), jnp.float32),
            pltpu.VMEM((NEG_PER_CHUNK, W), jnp.float32),
            pltpu.VMEM((CHUNK,), jnp.float32),
            pltpu.VMEM((NEG_PER_CHUNK,), jnp.float32),
            pltpu.SemaphoreType.DMA,
        ],
        compiler_params=pltpu.CompilerParams(use_tc_tiling_on_sc=False,
                                             needs_layout_passes=False),
    )(user_ids, pos_sub_ids, neg_ids_flat, user_emb_w, sub_emb_w)


def kernel(user_ids, pos_sub_ids, neg_sub_ids, user_emb, sub_emb):
    uid = user_ids.astype(jnp.int32)
    pid = pos_sub_ids.astype(jnp.int32)
    nid = neg_sub_ids.astype(jnp.int32).reshape(B * K)
    ue = jnp.pad(user_emb, ((0, 0), (0, W - D)))
    se = jnp.pad(sub_emb, ((0, 0), (0, W - D)))
    pos_flat, neg_flat = _sc_forward(uid, pid, nid, ue, se)
    return (pos_flat, neg_flat.reshape(B, K))


# speedup vs baseline: 1.1491x; 1.1246x over previous
import jax
import jax.numpy as jnp
from jax import lax
from jax.experimental import pallas as pl
from jax.experimental.pallas import tpu as pltpu
from jax.experimental.pallas import tpu_sc as plsc

NUM_USERS = 1000000
NUM_SUBS = 100000
D = 32
B = 16384
K = 20

NC = 2
NS = 16
NW = NC * NS
BPW = B // NW
CHUNK = 64
NCHUNK = BPW // CHUNK
GATHER_N = 128
NEG_PER_CHUNK = CHUNK * K
NEG_GATHERS = NEG_PER_CHUNK // GATHER_N
GROUPS = CHUNK // 16


def _sc_body(uid_hbm, pid_hbm, nid_hbm, user_emb, sub_emb,
             pos_out, neg_out,
             idx_u0, idx_p0, idx_n0, u_v0, sp_v0, sn_v0, pos_v0, neg_v0,
             idx_u1, idx_p1, idx_n1, u_v1, sp_v1, sn_v1, pos_v1, neg_v1,
             sem0, sem1):
    wid = lax.axis_index("s") * NC + lax.axis_index("c")
    bufs = ((idx_u0, idx_p0, idx_n0, u_v0, sp_v0, sn_v0, pos_v0, neg_v0, sem0),
            (idx_u1, idx_p1, idx_n1, u_v1, sp_v1, sn_v1, pos_v1, neg_v1, sem1))

    def stage_and_fire(c):
        idx_u, idx_p, idx_n, u_v, sp_v, sn_v, _, _, sem = bufs[c % 2]
        base = wid * BPW + c * CHUNK
        pltpu.sync_copy(uid_hbm.at[pl.ds(base, CHUNK)], idx_u)
        pltpu.sync_copy(pid_hbm.at[pl.ds(base, CHUNK)], idx_p)
        pltpu.sync_copy(nid_hbm.at[pl.ds(base * K, NEG_PER_CHUNK)], idx_n)
        cps = [pltpu.async_copy(user_emb.at[idx_u], u_v, sem),
               pltpu.async_copy(sub_emb.at[idx_p], sp_v, sem)]
        for j in range(NEG_GATHERS):
            cps.append(pltpu.async_copy(
                sub_emb.at[idx_n.at[pl.ds(j * GATHER_N, GATHER_N)]],
                sn_v.at[pl.ds(j * GATHER_N, GATHER_N), :], sem))
        return cps

    def compute_and_emit(c, cps):
        _, _, _, u_v, sp_v, sn_v, pos_v, neg_v, _ = bufs[c % 2]
        base = wid * BPW + c * CHUNK
        for cp in cps:
            cp.wait()

        def group_body(g, gcarry):
            rows = g * 16 + lax.iota(jnp.int32, 16)
            rows_k = rows * K
            zero = jnp.zeros((16,), jnp.float32)

            lanes = lax.iota(jnp.int32, 16)

            def d_body(d, accs):
                cold = (lanes + d) & (D - 1)
                u_d = plsc.load_gather(u_v, [rows, cold])
                p_d = plsc.load_gather(sp_v, [rows, cold])
                out = [accs[0] + u_d * p_d]
                for k in range(K):
                    n_d = plsc.load_gather(sn_v, [rows_k + k, cold])
                    out.append(accs[k + 1] + u_d * n_d)
                return tuple(out)

            accs = lax.fori_loop(0, D, d_body, (zero,) * (K + 1))
            pos_v[pl.ds(g * 16, 16)] = accs[0]
            for k in range(K):
                plsc.store_scatter(neg_v, [rows_k + k], accs[k + 1])
            return gcarry

        lax.fori_loop(0, GROUPS, group_body, 0)

        pltpu.sync_copy(pos_v, pos_out.at[pl.ds(base, CHUNK)])
        pltpu.sync_copy(neg_v, neg_out.at[pl.ds(base * K, NEG_PER_CHUNK)])

    inflight = stage_and_fire(0)
    for c in range(NCHUNK):
        nxt = stage_and_fire(c + 1) if c + 1 < NCHUNK else None
        compute_and_emit(c, inflight)
        inflight = nxt


@jax.jit
def _sc_forward(user_ids, pos_sub_ids, neg_ids_2d, user_emb, sub_emb):
    mesh = plsc.VectorSubcoreMesh(core_axis_name="c", subcore_axis_name="s")
    return pl.kernel(
        _sc_body,
        out_type=(jax.ShapeDtypeStruct((B,), jnp.float32),
                  jax.ShapeDtypeStruct((B * K,), jnp.float32)),
        mesh=mesh,
        scratch_types=[
            pltpu.VMEM((CHUNK,), jnp.int32),
            pltpu.VMEM((CHUNK,), jnp.int32),
            pltpu.VMEM((NEG_PER_CHUNK,), jnp.int32),
            pltpu.VMEM((CHUNK, D), jnp.float32),
            pltpu.VMEM((CHUNK, D), jnp.float32),
            pltpu.VMEM((NEG_PER_CHUNK, D), jnp.float32),
            pltpu.VMEM((CHUNK,), jnp.float32),
            pltpu.VMEM((NEG_PER_CHUNK,), jnp.float32),
            pltpu.VMEM((CHUNK,), jnp.int32),
            pltpu.VMEM((CHUNK,), jnp.int32),
            pltpu.VMEM((NEG_PER_CHUNK,), jnp.int32),
            pltpu.VMEM((CHUNK, D), jnp.float32),
            pltpu.VMEM((CHUNK, D), jnp.float32),
            pltpu.VMEM((NEG_PER_CHUNK, D), jnp.float32),
            pltpu.VMEM((CHUNK,), jnp.float32),
            pltpu.VMEM((NEG_PER_CHUNK,), jnp.float32),
            pltpu.SemaphoreType.DMA,
            pltpu.SemaphoreType.DMA,
        ],
        compiler_params=pltpu.CompilerParams(use_tc_tiling_on_sc=False, needs_layout_passes=False),
    )(user_ids, pos_sub_ids, neg_ids_2d, user_emb, sub_emb)


def kernel(user_ids, pos_sub_ids, neg_sub_ids, user_emb, sub_emb):
    uid = user_ids.astype(jnp.int32)
    pid = pos_sub_ids.astype(jnp.int32)
    nid = neg_sub_ids.astype(jnp.int32).reshape(B * K)
    pos_flat, neg_flat = _sc_forward(uid, pid, nid, user_emb, sub_emb)
    return (pos_flat, neg_flat.reshape(B, K))
